# baseline (speedup 1.0000x reference)
import functools

import jax
import jax.numpy as jnp
from jax import lax
from jax.experimental import pallas as pl
from jax.experimental.pallas import tpu as pltpu
from jax.experimental.pallas import tpu_sc as plsc

B = 32
N = 16384
NCH = 9
NCORES = 2
NSUB = 16
CHUNK = 2048
NCHUNK = N // CHUNK
GROUPS = CHUNK // 16
BIG = 1.0e6


def _sc_topk_body(pcld_hbm, kpts_hbm, cpt_hbm, seg_hbm, out_hbm,
                  kbuf, cbuf, pbuf, sbuf,
                  topk, topx, topy, topz,
                  pendk, pendx, pendy, pendz,
                  perm_scr, cnt_scr):
    wid = lax.axis_index("s") * NCORES + lax.axis_index("c")
    lane = lax.iota(jnp.int32, 16)
    inf16 = jnp.full((16,), jnp.inf, dtype=jnp.float32)
    zero16 = jnp.zeros((16,), dtype=jnp.float32)
    lt10 = lane < 10

    for c in range(NCH):
        topk[pl.ds(c * 16, 16)] = inf16
        topx[pl.ds(c * 16, 16)] = zero16
        topy[pl.ds(c * 16, 16)] = zero16
        topz[pl.ds(c * 16, 16)] = zero16

    kpts_base = wid * (N * 24)
    pnt_base = wid * (N * 3)
    seg_base = wid * (N * 2)

    def chunk_body(ci, thrs):
        off = ci * CHUNK
        pltpu.sync_copy(kpts_hbm.at[pl.ds(kpts_base + off * 24, CHUNK * 24)], kbuf)
        pltpu.sync_copy(cpt_hbm.at[pl.ds(pnt_base + off * 3, CHUNK * 3)], cbuf)
        pltpu.sync_copy(pcld_hbm.at[pl.ds(pnt_base + off * 3, CHUNK * 3)], pbuf)
        pltpu.sync_copy(seg_hbm.at[pl.ds(seg_base + off * 2, CHUNK * 2)], sbuf)

        def group_body(g, thrs):
            kb = g * (16 * 24) + lane * 24
            pb = g * (16 * 3) + lane * 3
            sb = g * (16 * 2) + lane * 2
            s0 = plsc.load_gather(sbuf, [sb])
            s1 = plsc.load_gather(sbuf, [sb + 1])
            bigm = jnp.where(s0 >= s1, BIG, 0.0).astype(jnp.float32)
            new_thrs = []
            for c in range(NCH):
                if c < 8:
                    ox = plsc.load_gather(kbuf, [kb + (c * 3 + 0)])
                    oy = plsc.load_gather(kbuf, [kb + (c * 3 + 1)])
                    oz = plsc.load_gather(kbuf, [kb + (c * 3 + 2)])
                else:
                    ox = plsc.load_gather(cbuf, [pb])
                    oy = plsc.load_gather(cbuf, [pb + 1])
                    oz = plsc.load_gather(cbuf, [pb + 2])
                key = ox * ox + oy * oy + oz * oz + bigm
                thr = thrs[c]
                m = key < thr

                def do_merge(c=c, key=key, m=m, ox=ox, oy=oy, oz=oz):
                    px = plsc.load_gather(pbuf, [pb])
                    py = plsc.load_gather(pbuf, [pb + 1])
                    pz = plsc.load_gather(pbuf, [pb + 2])
                    plsc.store_compressed(pendk.at[pl.ds(0, 16)], key, mask=m)
                    plsc.store_compressed(pendx.at[pl.ds(0, 16)], px + ox, mask=m)
                    plsc.store_compressed(pendy.at[pl.ds(0, 16)], py + oy, mask=m)
                    plsc.store_compressed(pendz.at[pl.ds(0, 16)], pz + oz, mask=m)
                    cnt = plsc.all_reduce_population_count(m)[0]
                    pendk[pl.ds(cnt, 16)] = inf16

                    def merge_round(pos):
                        gidx = jnp.maximum(lane - 10, 0) + pos
                        tk = topk[pl.ds(c * 16, 16)]
                        pk = plsc.load_gather(pendk, [gidx])
                        mk = jnp.where(lt10, tk, pk)
                        sk, pm = plsc.sort_key_val(mk, lane)
                        for (top_r, pend_r) in ((topx, pendx), (topy, pendy),
                                                (topz, pendz)):
                            tv = top_r[pl.ds(c * 16, 16)]
                            pv = plsc.load_gather(pend_r, [gidx])
                            perm_scr[...] = jnp.where(lt10, tv, pv)
                            top_r[pl.ds(c * 16, 16)] = plsc.load_gather(perm_scr, [pm])
                        topk[pl.ds(c * 16, 16)] = jnp.where(lt10, sk, inf16)
                        return jnp.full((16,), sk[9], dtype=jnp.float32)

                    t1 = merge_round(0)
                    t2 = lax.cond(cnt > 6, lambda: merge_round(6), lambda: t1)
                    t3 = lax.cond(cnt > 12, lambda: merge_round(12), lambda: t2)
                    return t3

                thr_new = lax.cond(jnp.any(m), do_merge, lambda thr=thr: thr)
                new_thrs.append(thr_new)
            return tuple(new_thrs)

        return lax.fori_loop(0, GROUPS, group_body, thrs)

    thrs0 = tuple(inf16 for _ in range(NCH))
    lax.fori_loop(0, NCHUNK, chunk_body, thrs0)

    out_base = wid * (NCH * 3 * 16)
    for c in range(NCH):
        pltpu.sync_copy(topx.at[pl.ds(c * 16, 16)],
                        out_hbm.at[pl.ds(out_base + (c * 3 + 0) * 16, 16)])
        pltpu.sync_copy(topy.at[pl.ds(c * 16, 16)],
                        out_hbm.at[pl.ds(out_base + (c * 3 + 1) * 16, 16)])
        pltpu.sync_copy(topz.at[pl.ds(c * 16, 16)],
                        out_hbm.at[pl.ds(out_base + (c * 3 + 2) * 16, 16)])


_sc_topk = pl.kernel(
    _sc_topk_body,
    out_type=jax.ShapeDtypeStruct((B * NCH * 3 * 16,), jnp.float32),
    mesh=plsc.VectorSubcoreMesh(core_axis_name="c", subcore_axis_name="s",
                                num_cores=NCORES, num_subcores=NSUB),
    compiler_params=pltpu.CompilerParams(needs_layout_passes=False),
    scratch_types=[
        pltpu.VMEM((CHUNK * 24,), jnp.float32),
        pltpu.VMEM((CHUNK * 3,), jnp.float32),
        pltpu.VMEM((CHUNK * 3,), jnp.float32),
        pltpu.VMEM((CHUNK * 2,), jnp.float32),
        pltpu.VMEM((NCH * 16,), jnp.float32),
        pltpu.VMEM((NCH * 16,), jnp.float32),
        pltpu.VMEM((NCH * 16,), jnp.float32),
        pltpu.VMEM((NCH * 16,), jnp.float32),
        pltpu.VMEM((32,), jnp.float32),
        pltpu.VMEM((32,), jnp.float32),
        pltpu.VMEM((32,), jnp.float32),
        pltpu.VMEM((32,), jnp.float32),
        pltpu.VMEM((16,), jnp.float32),
        pltpu.VMEM((16,), jnp.int32),
    ],
)


def _tc_pose_kernel(cx_ref, cy_ref, cz_ref, mx_ref, my_ref, mz_ref,
                    r_ref, t_ref, vx_ref, vy_ref, vz_ref):
    kvalid = (lax.broadcasted_iota(jnp.int32, (B, NCH, 16), 2) < 10)
    valid = kvalid.astype(jnp.float32)

    def cluster(c):
        mean = jnp.sum(c * valid, axis=2, keepdims=True) / 10.0
        dev = (c - mean) * valid
        std = jnp.sqrt(jnp.sum(dev * dev, axis=2, keepdims=True) / 10.0)
        m = ((jnp.abs(c - mean) <= std) & kvalid).astype(jnp.float32)
        n = jnp.sum(m, axis=2)
        return jnp.sum(c * m, axis=2) / (n + 1e-6)

    vx = cluster(cx_ref[...])
    vy = cluster(cy_ref[...])
    vz = cluster(cz_ref[...])
    vx_ref[...] = vx
    vy_ref[...] = vy
    vz_ref[...] = vz

    mx, my, mz = mx_ref[...], my_ref[...], mz_ref[...]
    inv_n = 1.0 / NCH

    def center(a):
        return a - jnp.sum(a, axis=1, keepdims=True) * inv_n

    amx, amy, amz = center(mx), center(my), center(mz)
    bmx, bmy, bmz = center(vx), center(vy), center(vz)

    def S(a, b):
        return jnp.sum(a * b, axis=1, keepdims=True) * inv_n

    sxx, sxy, sxz = S(amx, bmx), S(amx, bmy), S(amx, bmz)
    syx, syy, syz = S(amy, bmx), S(amy, bmy), S(amy, bmz)
    szx, szy, szz = S(amz, bmx), S(amz, bmy), S(amz, bmz)

    a = [[None] * 4 for _ in range(4)]
    a[0][0] = sxx + syy + szz
    a[0][1] = syz - szy
    a[0][2] = szx - sxz
    a[0][3] = sxy - syx
    a[1][1] = sxx - syy - szz
    a[1][2] = sxy + syx
    a[1][3] = szx + sxz
    a[2][2] = -sxx + syy - szz
    a[2][3] = syz + szy
    a[3][3] = -sxx - syy + szz

    one = jnp.ones_like(sxx)
    zero = jnp.zeros_like(sxx)
    v = [[one if i == j else zero for j in range(4)] for i in range(4)]
    for _sweep in range(8):
        for (p, q) in [(0, 1), (0, 2), (0, 3), (1, 2), (1, 3), (2, 3)]:
            app, aqq, apq = a[p][p], a[q][q], a[p][q]
            small = jnp.abs(apq) <= 1e-30
            denom = jnp.where(small, 1.0, 2.0 * apq)
            theta = (aqq - app) / denom
            t = jnp.sign(theta) / (jnp.abs(theta) + jnp.sqrt(theta * theta + 1.0))
            t = jnp.where(small, 0.0, t)
            tt1 = t * t + 1.0
            cth = 1.0 / jnp.sqrt(tt1)
            cth = cth * (1.5 - 0.5 * tt1 * cth * cth)
            sth = t * cth
            new = {}
            for k in range(4):
                if k in (p, q):
                    continue
                akp = a[min(k, p)][max(k, p)]
                akq = a[min(k, q)][max(k, q)]
                new[(min(k, p), max(k, p))] = cth * akp - sth * akq
                new[(min(k, q), max(k, q))] = sth * akp + cth * akq
            a[p][p] = cth * cth * app - 2.0 * sth * cth * apq + sth * sth * aqq
            a[q][q] = sth * sth * app + 2.0 * sth * cth * apq + cth * cth * aqq
            a[p][q] = zero
            for (i, j), val in new.items():
                a[i][j] = val
            for k in range(4):
                vkp, vkq = v[k][p], v[k][q]
                v[k][p] = cth * vkp - sth * vkq
                v[k][q] = sth * vkp + cth * vkq

    d = [a[i][i] for i in range(4)]

    def pick(da, va, db, vb):
        cnd = da >= db
        return (jnp.where(cnd, da, db),
                [jnp.where(cnd, x, y) for x, y in zip(va, vb)])

    cols = [[v[i][j] for i in range(4)] for j in range(4)]
    d01, v01 = pick(d[0], cols[0], d[1], cols[1])
    d23, v23 = pick(d[2], cols[2], d[3], cols[3])
    _, (qw, qx, qy, qz) = pick(d01, v01, d23, v23)

    qn = qw * qw + qx * qx + qy * qy + qz * qz
    qs = 1.0 / jnp.sqrt(qn)
    qs = qs * (1.5 - 0.5 * qn * qs * qs)
    qw, qx, qy, qz = qw * qs, qx * qs, qy * qs, qz * qs

    r00 = 1.0 - 2.0 * (qy * qy + qz * qz)
    r01 = 2.0 * (qx * qy - qz * qw)
    r02 = 2.0 * (qx * qz + qy * qw)
    r10 = 2.0 * (qx * qy + qz * qw)
    r11 = 1.0 - 2.0 * (qx * qx + qz * qz)
    r12 = 2.0 * (qy * qz - qx * qw)
    r20 = 2.0 * (qx * qz - qy * qw)
    r21 = 2.0 * (qy * qz + qx * qw)
    r22 = 1.0 - 2.0 * (qx * qx + qy * qy)

    cax = jnp.sum(mx, axis=1, keepdims=True) * inv_n
    cay = jnp.sum(my, axis=1, keepdims=True) * inv_n
    caz = jnp.sum(mz, axis=1, keepdims=True) * inv_n
    cbx = jnp.sum(vx, axis=1, keepdims=True) * inv_n
    cby = jnp.sum(vy, axis=1, keepdims=True) * inv_n
    cbz = jnp.sum(vz, axis=1, keepdims=True) * inv_n
    tx = cbx - (r00 * cax + r01 * cay + r02 * caz)
    ty = cby - (r10 * cax + r11 * cay + r12 * caz)
    tz = cbz - (r20 * cax + r21 * cay + r22 * caz)

    r_ref[...] = jnp.concatenate(
        [r00, r01, r02, r10, r11, r12, r20, r21, r22], axis=1)
    t_ref[...] = jnp.concatenate([tx, ty, tz], axis=1)


_tc_pose = pl.pallas_call(
    _tc_pose_kernel,
    out_shape=(
        jax.ShapeDtypeStruct((B, 9), jnp.float32),
        jax.ShapeDtypeStruct((B, 3), jnp.float32),
        jax.ShapeDtypeStruct((B, NCH), jnp.float32),
        jax.ShapeDtypeStruct((B, NCH), jnp.float32),
        jax.ShapeDtypeStruct((B, NCH), jnp.float32),
    ),
)


def kernel(pcld_input, kpts_pre_input, cpt_pre_input, seg_pre_input,
           mesh_kpts_input):
    cands = _sc_topk(
        pcld_input.reshape(-1),
        kpts_pre_input.reshape(-1),
        cpt_pre_input.reshape(-1),
        seg_pre_input.reshape(-1),
    ).reshape(B, NCH, 3, 16)
    cx = cands[:, :, 0, :]
    cy = cands[:, :, 1, :]
    cz = cands[:, :, 2, :]
    mx = mesh_kpts_input[:, :, 0]
    my = mesh_kpts_input[:, :, 1]
    mz = mesh_kpts_input[:, :, 2]
    rflat, t, vx, vy, vz = _tc_pose(cx, cy, cz, mx, my, mz)
    batch_R = rflat.reshape(B, 3, 3)
    voted = jnp.stack([vx, vy, vz], axis=-1)
    return (batch_R, t, voted)


# speedup vs baseline: 10.4550x; 10.4550x over previous
import functools

import jax
import jax.numpy as jnp
from jax import lax
from jax.experimental import pallas as pl
from jax.experimental.pallas import tpu as pltpu
from jax.experimental.pallas import tpu_sc as plsc

B = 32
N = 16384
NCH = 9
NCORES = 2
NSUB = 16
CHUNK = 2048
NCHUNK = N // CHUNK
GROUPS = CHUNK // 16
BIG = 1.0e6


def _sc_topk_body(pcld_hbm, kpts_hbm, cpt_hbm, seg_hbm, out_hbm,
                  kbuf, cbuf, pbuf, sbuf,
                  topk, topx, topy, topz,
                  pendk, pendx, pendy, pendz,
                  perm_scr, cnt_scr):
    wid = lax.axis_index("s") * NCORES + lax.axis_index("c")
    lane = lax.iota(jnp.int32, 16)
    inf16 = jnp.full((16,), jnp.inf, dtype=jnp.float32)
    zero16 = jnp.zeros((16,), dtype=jnp.float32)
    lt10 = lane < 10

    for c in range(NCH):
        topk[pl.ds(c * 16, 16)] = inf16
        topx[pl.ds(c * 16, 16)] = zero16
        topy[pl.ds(c * 16, 16)] = zero16
        topz[pl.ds(c * 16, 16)] = zero16

    bh = wid // 8
    bl = wid % 8
    NHC = CHUNK // 128

    def chunk_body(ci, thrs):
        nh0 = ci * NHC
        for comp in range(3):
            pltpu.sync_copy(
                kpts_hbm.at[pl.ds(((wid * 3 + comp) * 128 + nh0) * 1024, NHC * 1024)],
                kbuf.at[pl.ds(comp * (NHC * 1024), NHC * 1024)])
            pltpu.sync_copy(
                pcld_hbm.at[pl.ds(((comp * 4 + bh) * 128 + nh0) * 1024, NHC * 1024)],
                pbuf.at[pl.ds(comp * (NHC * 1024), NHC * 1024)])
            pltpu.sync_copy(
                cpt_hbm.at[pl.ds((wid * 3 + comp) * N + ci * CHUNK, CHUNK)],
                cbuf.at[pl.ds(comp * CHUNK, CHUNK)])
        pltpu.sync_copy(seg_hbm.at[pl.ds((wid * 128 + nh0) * 256, NHC * 256)],
                        sbuf)

        def group_body(g, thrs):
            nh = g // 8
            l0 = (g % 8) * 16
            s0 = sbuf[pl.ds(nh * 256 + l0, 16)]
            s1 = sbuf[pl.ds(nh * 256 + 128 + l0, 16)]
            bigm = jnp.where(s0 >= s1, BIG, 0.0).astype(jnp.float32)
            new_thrs = []
            for c in range(NCH):
                if c < 8:
                    ox = kbuf[pl.ds(0 * (NHC * 1024) + nh * 1024 + c * 128 + l0, 16)]
                    oy = kbuf[pl.ds(1 * (NHC * 1024) + nh * 1024 + c * 128 + l0, 16)]
                    oz = kbuf[pl.ds(2 * (NHC * 1024) + nh * 1024 + c * 128 + l0, 16)]
                else:
                    ox = cbuf[pl.ds(0 * CHUNK + nh * 128 + l0, 16)]
                    oy = cbuf[pl.ds(1 * CHUNK + nh * 128 + l0, 16)]
                    oz = cbuf[pl.ds(2 * CHUNK + nh * 128 + l0, 16)]
                key = ox * ox + oy * oy + oz * oz + bigm
                thr = thrs[c]
                m = key < thr

                def do_merge(c=c, key=key, m=m, ox=ox, oy=oy, oz=oz,
                             nh=nh, l0=l0):
                    px = pbuf[pl.ds(0 * (NHC * 1024) + nh * 1024 + bl * 128 + l0, 16)]
                    py = pbuf[pl.ds(1 * (NHC * 1024) + nh * 1024 + bl * 128 + l0, 16)]
                    pz = pbuf[pl.ds(2 * (NHC * 1024) + nh * 1024 + bl * 128 + l0, 16)]
                    plsc.store_compressed(pendk.at[pl.ds(0, 16)], key, mask=m)
                    plsc.store_compressed(pendx.at[pl.ds(0, 16)], px + ox, mask=m)
                    plsc.store_compressed(pendy.at[pl.ds(0, 16)], py + oy, mask=m)
                    plsc.store_compressed(pendz.at[pl.ds(0, 16)], pz + oz, mask=m)
                    cnt = plsc.all_reduce_population_count(m)[0]
                    pendk[pl.ds(cnt, 16)] = inf16

                    def merge_round(pos):
                        gidx = jnp.maximum(lane - 10, 0) + pos
                        tk = topk[pl.ds(c * 16, 16)]
                        pk = plsc.load_gather(pendk, [gidx])
                        mk = jnp.where(lt10, tk, pk)
                        sk, pm = plsc.sort_key_val(mk, lane)
                        for (top_r, pend_r) in ((topx, pendx), (topy, pendy),
                                                (topz, pendz)):
                            tv = top_r[pl.ds(c * 16, 16)]
                            pv = plsc.load_gather(pend_r, [gidx])
                            perm_scr[...] = jnp.where(lt10, tv, pv)
                            top_r[pl.ds(c * 16, 16)] = plsc.load_gather(perm_scr, [pm])
                        topk[pl.ds(c * 16, 16)] = jnp.where(lt10, sk, inf16)
                        return jnp.full((16,), sk[9], dtype=jnp.float32)

                    t1 = merge_round(0)
                    t2 = lax.cond(cnt > 6, lambda: merge_round(6), lambda: t1)
                    t3 = lax.cond(cnt > 12, lambda: merge_round(12), lambda: t2)
                    return t3

                thr_new = lax.cond(jnp.any(m), do_merge, lambda thr=thr: thr)
                new_thrs.append(thr_new)
            return tuple(new_thrs)

        return lax.fori_loop(0, GROUPS, group_body, thrs)

    thrs0 = tuple(inf16 for _ in range(NCH))
    lax.fori_loop(0, NCHUNK, chunk_body, thrs0)

    out_base = wid * (NCH * 3 * 16)
    for c in range(NCH):
        pltpu.sync_copy(topx.at[pl.ds(c * 16, 16)],
                        out_hbm.at[pl.ds(out_base + (c * 3 + 0) * 16, 16)])
        pltpu.sync_copy(topy.at[pl.ds(c * 16, 16)],
                        out_hbm.at[pl.ds(out_base + (c * 3 + 1) * 16, 16)])
        pltpu.sync_copy(topz.at[pl.ds(c * 16, 16)],
                        out_hbm.at[pl.ds(out_base + (c * 3 + 2) * 16, 16)])


_sc_topk = pl.kernel(
    _sc_topk_body,
    out_type=jax.ShapeDtypeStruct((B * NCH * 3 * 16,), jnp.float32),
    mesh=plsc.VectorSubcoreMesh(core_axis_name="c", subcore_axis_name="s",
                                num_cores=NCORES, num_subcores=NSUB),
    compiler_params=pltpu.CompilerParams(needs_layout_passes=False),
    scratch_types=[
        pltpu.VMEM((CHUNK * 24,), jnp.float32),
        pltpu.VMEM((CHUNK * 3,), jnp.float32),
        pltpu.VMEM((CHUNK * 24,), jnp.float32),
        pltpu.VMEM((CHUNK * 2,), jnp.float32),
        pltpu.VMEM((NCH * 16,), jnp.float32),
        pltpu.VMEM((NCH * 16,), jnp.float32),
        pltpu.VMEM((NCH * 16,), jnp.float32),
        pltpu.VMEM((NCH * 16,), jnp.float32),
        pltpu.VMEM((32,), jnp.float32),
        pltpu.VMEM((32,), jnp.float32),
        pltpu.VMEM((32,), jnp.float32),
        pltpu.VMEM((32,), jnp.float32),
        pltpu.VMEM((16,), jnp.float32),
        pltpu.VMEM((16,), jnp.int32),
    ],
)


def _tc_pose_kernel(cx_ref, cy_ref, cz_ref, mx_ref, my_ref, mz_ref,
                    r_ref, t_ref, vx_ref, vy_ref, vz_ref):
    kvalid = (lax.broadcasted_iota(jnp.int32, (B, NCH, 16), 2) < 10)
    valid = kvalid.astype(jnp.float32)

    def cluster(c):
        mean = jnp.sum(c * valid, axis=2, keepdims=True) / 10.0
        dev = (c - mean) * valid
        std = jnp.sqrt(jnp.sum(dev * dev, axis=2, keepdims=True) / 10.0)
        m = ((jnp.abs(c - mean) <= std) & kvalid).astype(jnp.float32)
        n = jnp.sum(m, axis=2)
        return jnp.sum(c * m, axis=2) / (n + 1e-6)

    vx = cluster(cx_ref[...])
    vy = cluster(cy_ref[...])
    vz = cluster(cz_ref[...])
    vx_ref[...] = vx
    vy_ref[...] = vy
    vz_ref[...] = vz

    mx, my, mz = mx_ref[...], my_ref[...], mz_ref[...]
    inv_n = 1.0 / NCH

    def center(a):
        return a - jnp.sum(a, axis=1, keepdims=True) * inv_n

    amx, amy, amz = center(mx), center(my), center(mz)
    bmx, bmy, bmz = center(vx), center(vy), center(vz)

    def S(a, b):
        return jnp.sum(a * b, axis=1, keepdims=True) * inv_n

    sxx, sxy, sxz = S(amx, bmx), S(amx, bmy), S(amx, bmz)
    syx, syy, syz = S(amy, bmx), S(amy, bmy), S(amy, bmz)
    szx, szy, szz = S(amz, bmx), S(amz, bmy), S(amz, bmz)

    a = [[None] * 4 for _ in range(4)]
    a[0][0] = sxx + syy + szz
    a[0][1] = syz - szy
    a[0][2] = szx - sxz
    a[0][3] = sxy - syx
    a[1][1] = sxx - syy - szz
    a[1][2] = sxy + syx
    a[1][3] = szx + sxz
    a[2][2] = -sxx + syy - szz
    a[2][3] = syz + szy
    a[3][3] = -sxx - syy + szz

    one = jnp.ones_like(sxx)
    zero = jnp.zeros_like(sxx)
    v = [[one if i == j else zero for j in range(4)] for i in range(4)]
    for _sweep in range(8):
        for (p, q) in [(0, 1), (0, 2), (0, 3), (1, 2), (1, 3), (2, 3)]:
            app, aqq, apq = a[p][p], a[q][q], a[p][q]
            small = jnp.abs(apq) <= 1e-30
            denom = jnp.where(small, 1.0, 2.0 * apq)
            theta = (aqq - app) / denom
            t = jnp.sign(theta) / (jnp.abs(theta) + jnp.sqrt(theta * theta + 1.0))
            t = jnp.where(small, 0.0, t)
            tt1 = t * t + 1.0
            cth = 1.0 / jnp.sqrt(tt1)
            cth = cth * (1.5 - 0.5 * tt1 * cth * cth)
            sth = t * cth
            new = {}
            for k in range(4):
                if k in (p, q):
                    continue
                akp = a[min(k, p)][max(k, p)]
                akq = a[min(k, q)][max(k, q)]
                new[(min(k, p), max(k, p))] = cth * akp - sth * akq
                new[(min(k, q), max(k, q))] = sth * akp + cth * akq
            a[p][p] = cth * cth * app - 2.0 * sth * cth * apq + sth * sth * aqq
            a[q][q] = sth * sth * app + 2.0 * sth * cth * apq + cth * cth * aqq
            a[p][q] = zero
            for (i, j), val in new.items():
                a[i][j] = val
            for k in range(4):
                vkp, vkq = v[k][p], v[k][q]
                v[k][p] = cth * vkp - sth * vkq
                v[k][q] = sth * vkp + cth * vkq

    d = [a[i][i] for i in range(4)]

    def pick(da, va, db, vb):
        cnd = da >= db
        return (jnp.where(cnd, da, db),
                [jnp.where(cnd, x, y) for x, y in zip(va, vb)])

    cols = [[v[i][j] for i in range(4)] for j in range(4)]
    d01, v01 = pick(d[0], cols[0], d[1], cols[1])
    d23, v23 = pick(d[2], cols[2], d[3], cols[3])
    _, (qw, qx, qy, qz) = pick(d01, v01, d23, v23)

    qn = qw * qw + qx * qx + qy * qy + qz * qz
    qs = 1.0 / jnp.sqrt(qn)
    qs = qs * (1.5 - 0.5 * qn * qs * qs)
    qw, qx, qy, qz = qw * qs, qx * qs, qy * qs, qz * qs

    r00 = 1.0 - 2.0 * (qy * qy + qz * qz)
    r01 = 2.0 * (qx * qy - qz * qw)
    r02 = 2.0 * (qx * qz + qy * qw)
    r10 = 2.0 * (qx * qy + qz * qw)
    r11 = 1.0 - 2.0 * (qx * qx + qz * qz)
    r12 = 2.0 * (qy * qz - qx * qw)
    r20 = 2.0 * (qx * qz - qy * qw)
    r21 = 2.0 * (qy * qz + qx * qw)
    r22 = 1.0 - 2.0 * (qx * qx + qy * qy)

    cax = jnp.sum(mx, axis=1, keepdims=True) * inv_n
    cay = jnp.sum(my, axis=1, keepdims=True) * inv_n
    caz = jnp.sum(mz, axis=1, keepdims=True) * inv_n
    cbx = jnp.sum(vx, axis=1, keepdims=True) * inv_n
    cby = jnp.sum(vy, axis=1, keepdims=True) * inv_n
    cbz = jnp.sum(vz, axis=1, keepdims=True) * inv_n
    tx = cbx - (r00 * cax + r01 * cay + r02 * caz)
    ty = cby - (r10 * cax + r11 * cay + r12 * caz)
    tz = cbz - (r20 * cax + r21 * cay + r22 * caz)

    r_ref[...] = jnp.concatenate(
        [r00, r01, r02, r10, r11, r12, r20, r21, r22], axis=1)
    t_ref[...] = jnp.concatenate([tx, ty, tz], axis=1)


_tc_pose = pl.pallas_call(
    _tc_pose_kernel,
    out_shape=(
        jax.ShapeDtypeStruct((B, 9), jnp.float32),
        jax.ShapeDtypeStruct((B, 3), jnp.float32),
        jax.ShapeDtypeStruct((B, NCH), jnp.float32),
        jax.ShapeDtypeStruct((B, NCH), jnp.float32),
        jax.ShapeDtypeStruct((B, NCH), jnp.float32),
    ),
)


def kernel(pcld_input, kpts_pre_input, cpt_pre_input, seg_pre_input,
           mesh_kpts_input):
    kflat = (kpts_pre_input.transpose(0, 3, 1, 2)
             .reshape(B, 3, 128, 128, 8)
             .transpose(0, 1, 2, 4, 3)
             .reshape(-1))
    pflat = (pcld_input.transpose(2, 0, 1)
             .reshape(3, 4, 8, 128, 128)
             .transpose(0, 1, 3, 2, 4)
             .reshape(-1))
    cflat = (cpt_pre_input.reshape(B, N, 3)
             .transpose(0, 2, 1)
             .reshape(-1))
    sflat = (seg_pre_input.reshape(B, 128, 128, 2)
             .transpose(0, 1, 3, 2)
             .reshape(-1))
    cands = _sc_topk(pflat, kflat, cflat, sflat).reshape(B, NCH, 3, 16)
    cx = cands[:, :, 0, :]
    cy = cands[:, :, 1, :]
    cz = cands[:, :, 2, :]
    mx = mesh_kpts_input[:, :, 0]
    my = mesh_kpts_input[:, :, 1]
    mz = mesh_kpts_input[:, :, 2]
    rflat, t, vx, vy, vz = _tc_pose(cx, cy, cz, mx, my, mz)
    batch_R = rflat.reshape(B, 3, 3)
    voted = jnp.stack([vx, vy, vz], axis=-1)
    return (batch_R, t, voted)


# speedup vs baseline: 23.9931x; 2.2949x over previous
import functools

import jax
import jax.numpy as jnp
from jax import lax
from jax.experimental import pallas as pl
from jax.experimental.pallas import tpu as pltpu
from jax.experimental.pallas import tpu_sc as plsc

B = 32
N = 16384
NCH = 9
NCORES = 2
NSUB = 16
CHUNK = 2048
NCHUNK = N // CHUNK
GROUPS = CHUNK // 16
BIG = 1.0e6


def _sc_topk_body(pcld_hbm, kpts_hbm, cpt_hbm, seg_hbm, out_hbm,
                  kbuf, cbuf, pbuf, sbuf,
                  topk, topx, topy, topz,
                  pendk, pendx, pendy, pendz,
                  perm_scr, cnt_scr):
    wid = lax.axis_index("s") * NCORES + lax.axis_index("c")
    lane = lax.iota(jnp.int32, 16)
    inf16 = jnp.full((16,), jnp.inf, dtype=jnp.float32)
    zero16 = jnp.zeros((16,), dtype=jnp.float32)
    lt10 = lane < 10

    for c in range(NCH):
        topk[pl.ds(c * 16, 16)] = inf16
        topx[pl.ds(c * 16, 16)] = zero16
        topy[pl.ds(c * 16, 16)] = zero16
        topz[pl.ds(c * 16, 16)] = zero16

    bh = wid // 8
    bl = wid % 8
    NHC = CHUNK // 128

    def chunk_body(ci, thrs):
        nh0 = ci * NHC
        for comp in range(3):
            pltpu.sync_copy(
                kpts_hbm.at[pl.ds(((wid * 3 + comp) * 128 + nh0) * 1024, NHC * 1024)],
                kbuf.at[pl.ds(comp * (NHC * 1024), NHC * 1024)])
            pltpu.sync_copy(
                pcld_hbm.at[pl.ds(((comp * 4 + bh) * 128 + nh0) * 1024, NHC * 1024)],
                pbuf.at[pl.ds(comp * (NHC * 1024), NHC * 1024)])
            pltpu.sync_copy(
                cpt_hbm.at[pl.ds((wid * 3 + comp) * N + ci * CHUNK, CHUNK)],
                cbuf.at[pl.ds(comp * CHUNK, CHUNK)])
        pltpu.sync_copy(seg_hbm.at[pl.ds((wid * 128 + nh0) * 256, NHC * 256)],
                        sbuf)

        def group_body(g, thrs):
            nh = g // 8
            l0 = (g % 8) * 16
            s0 = sbuf[pl.ds(nh * 256 + l0, 16)]
            s1 = sbuf[pl.ds(nh * 256 + 128 + l0, 16)]
            bigm = jnp.where(s0 >= s1, BIG, 0.0).astype(jnp.float32)
            keys = []
            masks = []
            anym = None
            for c in range(NCH):
                if c < 8:
                    ox = kbuf[pl.ds(0 * (NHC * 1024) + nh * 1024 + c * 128 + l0, 16)]
                    oy = kbuf[pl.ds(1 * (NHC * 1024) + nh * 1024 + c * 128 + l0, 16)]
                    oz = kbuf[pl.ds(2 * (NHC * 1024) + nh * 1024 + c * 128 + l0, 16)]
                else:
                    ox = cbuf[pl.ds(0 * CHUNK + nh * 128 + l0, 16)]
                    oy = cbuf[pl.ds(1 * CHUNK + nh * 128 + l0, 16)]
                    oz = cbuf[pl.ds(2 * CHUNK + nh * 128 + l0, 16)]
                key = ox * ox + oy * oy + oz * oz + bigm
                keys.append((key, ox, oy, oz))
                m = key < thrs[c]
                masks.append(m)
                anym = m if anym is None else (anym | m)

            def do_merges(nh=nh, l0=l0):
                px = pbuf[pl.ds(0 * (NHC * 1024) + nh * 1024 + bl * 128 + l0, 16)]
                py = pbuf[pl.ds(1 * (NHC * 1024) + nh * 1024 + bl * 128 + l0, 16)]
                pz = pbuf[pl.ds(2 * (NHC * 1024) + nh * 1024 + bl * 128 + l0, 16)]
                new_thrs = []
                for c in range(NCH):
                    key, ox, oy, oz = keys[c]
                    m = masks[c]

                    def do_merge(c=c, key=key, m=m, ox=ox, oy=oy, oz=oz):
                        plsc.store_compressed(pendk.at[pl.ds(0, 16)], key, mask=m)
                        plsc.store_compressed(pendx.at[pl.ds(0, 16)], px + ox, mask=m)
                        plsc.store_compressed(pendy.at[pl.ds(0, 16)], py + oy, mask=m)
                        plsc.store_compressed(pendz.at[pl.ds(0, 16)], pz + oz, mask=m)
                        cnt = plsc.all_reduce_population_count(m)[0]
                        pendk[pl.ds(cnt, 16)] = inf16

                        def merge_round(pos):
                            gidx = jnp.maximum(lane - 10, 0) + pos
                            tk = topk[pl.ds(c * 16, 16)]
                            pk = plsc.load_gather(pendk, [gidx])
                            mk = jnp.where(lt10, tk, pk)
                            sk, pm = plsc.sort_key_val(mk, lane)
                            for (top_r, pend_r) in ((topx, pendx), (topy, pendy),
                                                    (topz, pendz)):
                                tv = top_r[pl.ds(c * 16, 16)]
                                pv = plsc.load_gather(pend_r, [gidx])
                                perm_scr[...] = jnp.where(lt10, tv, pv)
                                top_r[pl.ds(c * 16, 16)] = plsc.load_gather(perm_scr, [pm])
                            topk[pl.ds(c * 16, 16)] = jnp.where(lt10, sk, inf16)
                            return jnp.full((16,), sk[9], dtype=jnp.float32)

                        t1 = merge_round(0)
                        t2 = lax.cond(cnt > 6, lambda: merge_round(6), lambda: t1)
                        t3 = lax.cond(cnt > 12, lambda: merge_round(12), lambda: t2)
                        return t3

                    thr_new = lax.cond(jnp.any(m), do_merge,
                                       lambda thr=thrs[c]: thr)
                    new_thrs.append(thr_new)
                return tuple(new_thrs)

            return lax.cond(jnp.any(anym), do_merges, lambda: thrs)

        return lax.fori_loop(0, GROUPS, group_body, thrs)

    thrs0 = tuple(inf16 for _ in range(NCH))
    lax.fori_loop(0, NCHUNK, chunk_body, thrs0)

    out_base = wid * (NCH * 3 * 16)
    for c in range(NCH):
        pltpu.sync_copy(topx.at[pl.ds(c * 16, 16)],
                        out_hbm.at[pl.ds(out_base + (c * 3 + 0) * 16, 16)])
        pltpu.sync_copy(topy.at[pl.ds(c * 16, 16)],
                        out_hbm.at[pl.ds(out_base + (c * 3 + 1) * 16, 16)])
        pltpu.sync_copy(topz.at[pl.ds(c * 16, 16)],
                        out_hbm.at[pl.ds(out_base + (c * 3 + 2) * 16, 16)])


_sc_topk = pl.kernel(
    _sc_topk_body,
    out_type=jax.ShapeDtypeStruct((B * NCH * 3 * 16,), jnp.float32),
    mesh=plsc.VectorSubcoreMesh(core_axis_name="c", subcore_axis_name="s",
                                num_cores=NCORES, num_subcores=NSUB),
    compiler_params=pltpu.CompilerParams(needs_layout_passes=False),
    scratch_types=[
        pltpu.VMEM((CHUNK * 24,), jnp.float32),
        pltpu.VMEM((CHUNK * 3,), jnp.float32),
        pltpu.VMEM((CHUNK * 24,), jnp.float32),
        pltpu.VMEM((CHUNK * 2,), jnp.float32),
        pltpu.VMEM((NCH * 16,), jnp.float32),
        pltpu.VMEM((NCH * 16,), jnp.float32),
        pltpu.VMEM((NCH * 16,), jnp.float32),
        pltpu.VMEM((NCH * 16,), jnp.float32),
        pltpu.VMEM((32,), jnp.float32),
        pltpu.VMEM((32,), jnp.float32),
        pltpu.VMEM((32,), jnp.float32),
        pltpu.VMEM((32,), jnp.float32),
        pltpu.VMEM((16,), jnp.float32),
        pltpu.VMEM((16,), jnp.int32),
    ],
)


def _tc_pose_kernel(cx_ref, cy_ref, cz_ref, mx_ref, my_ref, mz_ref,
                    r_ref, t_ref, vx_ref, vy_ref, vz_ref):
    kvalid = (lax.broadcasted_iota(jnp.int32, (B, NCH, 16), 2) < 10)
    valid = kvalid.astype(jnp.float32)

    def cluster(c):
        mean = jnp.sum(c * valid, axis=2, keepdims=True) / 10.0
        dev = (c - mean) * valid
        std = jnp.sqrt(jnp.sum(dev * dev, axis=2, keepdims=True) / 10.0)
        m = ((jnp.abs(c - mean) <= std) & kvalid).astype(jnp.float32)
        n = jnp.sum(m, axis=2)
        return jnp.sum(c * m, axis=2) / (n + 1e-6)

    vx = cluster(cx_ref[...])
    vy = cluster(cy_ref[...])
    vz = cluster(cz_ref[...])
    vx_ref[...] = vx
    vy_ref[...] = vy
    vz_ref[...] = vz

    mx, my, mz = mx_ref[...], my_ref[...], mz_ref[...]
    inv_n = 1.0 / NCH

    def center(a):
        return a - jnp.sum(a, axis=1, keepdims=True) * inv_n

    amx, amy, amz = center(mx), center(my), center(mz)
    bmx, bmy, bmz = center(vx), center(vy), center(vz)

    def S(a, b):
        return jnp.sum(a * b, axis=1, keepdims=True) * inv_n

    sxx, sxy, sxz = S(amx, bmx), S(amx, bmy), S(amx, bmz)
    syx, syy, syz = S(amy, bmx), S(amy, bmy), S(amy, bmz)
    szx, szy, szz = S(amz, bmx), S(amz, bmy), S(amz, bmz)

    a = [[None] * 4 for _ in range(4)]
    a[0][0] = sxx + syy + szz
    a[0][1] = syz - szy
    a[0][2] = szx - sxz
    a[0][3] = sxy - syx
    a[1][1] = sxx - syy - szz
    a[1][2] = sxy + syx
    a[1][3] = szx + sxz
    a[2][2] = -sxx + syy - szz
    a[2][3] = syz + szy
    a[3][3] = -sxx - syy + szz

    one = jnp.ones_like(sxx)
    zero = jnp.zeros_like(sxx)
    v = [[one if i == j else zero for j in range(4)] for i in range(4)]
    for _sweep in range(8):
        for (p, q) in [(0, 1), (0, 2), (0, 3), (1, 2), (1, 3), (2, 3)]:
            app, aqq, apq = a[p][p], a[q][q], a[p][q]
            small = jnp.abs(apq) <= 1e-30
            denom = jnp.where(small, 1.0, 2.0 * apq)
            theta = (aqq - app) / denom
            t = jnp.sign(theta) / (jnp.abs(theta) + jnp.sqrt(theta * theta + 1.0))
            t = jnp.where(small, 0.0, t)
            tt1 = t * t + 1.0
            cth = 1.0 / jnp.sqrt(tt1)
            cth = cth * (1.5 - 0.5 * tt1 * cth * cth)
            sth = t * cth
            new = {}
            for k in range(4):
                if k in (p, q):
                    continue
                akp = a[min(k, p)][max(k, p)]
                akq = a[min(k, q)][max(k, q)]
                new[(min(k, p), max(k, p))] = cth * akp - sth * akq
                new[(min(k, q), max(k, q))] = sth * akp + cth * akq
            a[p][p] = cth * cth * app - 2.0 * sth * cth * apq + sth * sth * aqq
            a[q][q] = sth * sth * app + 2.0 * sth * cth * apq + cth * cth * aqq
            a[p][q] = zero
            for (i, j), val in new.items():
                a[i][j] = val
            for k in range(4):
                vkp, vkq = v[k][p], v[k][q]
                v[k][p] = cth * vkp - sth * vkq
                v[k][q] = sth * vkp + cth * vkq

    d = [a[i][i] for i in range(4)]

    def pick(da, va, db, vb):
        cnd = da >= db
        return (jnp.where(cnd, da, db),
                [jnp.where(cnd, x, y) for x, y in zip(va, vb)])

    cols = [[v[i][j] for i in range(4)] for j in range(4)]
    d01, v01 = pick(d[0], cols[0], d[1], cols[1])
    d23, v23 = pick(d[2], cols[2], d[3], cols[3])
    _, (qw, qx, qy, qz) = pick(d01, v01, d23, v23)

    qn = qw * qw + qx * qx + qy * qy + qz * qz
    qs = 1.0 / jnp.sqrt(qn)
    qs = qs * (1.5 - 0.5 * qn * qs * qs)
    qw, qx, qy, qz = qw * qs, qx * qs, qy * qs, qz * qs

    r00 = 1.0 - 2.0 * (qy * qy + qz * qz)
    r01 = 2.0 * (qx * qy - qz * qw)
    r02 = 2.0 * (qx * qz + qy * qw)
    r10 = 2.0 * (qx * qy + qz * qw)
    r11 = 1.0 - 2.0 * (qx * qx + qz * qz)
    r12 = 2.0 * (qy * qz - qx * qw)
    r20 = 2.0 * (qx * qz - qy * qw)
    r21 = 2.0 * (qy * qz + qx * qw)
    r22 = 1.0 - 2.0 * (qx * qx + qy * qy)

    cax = jnp.sum(mx, axis=1, keepdims=True) * inv_n
    cay = jnp.sum(my, axis=1, keepdims=True) * inv_n
    caz = jnp.sum(mz, axis=1, keepdims=True) * inv_n
    cbx = jnp.sum(vx, axis=1, keepdims=True) * inv_n
    cby = jnp.sum(vy, axis=1, keepdims=True) * inv_n
    cbz = jnp.sum(vz, axis=1, keepdims=True) * inv_n
    tx = cbx - (r00 * cax + r01 * cay + r02 * caz)
    ty = cby - (r10 * cax + r11 * cay + r12 * caz)
    tz = cbz - (r20 * cax + r21 * cay + r22 * caz)

    r_ref[...] = jnp.concatenate(
        [r00, r01, r02, r10, r11, r12, r20, r21, r22], axis=1)
    t_ref[...] = jnp.concatenate([tx, ty, tz], axis=1)


_tc_pose = pl.pallas_call(
    _tc_pose_kernel,
    out_shape=(
        jax.ShapeDtypeStruct((B, 9), jnp.float32),
        jax.ShapeDtypeStruct((B, 3), jnp.float32),
        jax.ShapeDtypeStruct((B, NCH), jnp.float32),
        jax.ShapeDtypeStruct((B, NCH), jnp.float32),
        jax.ShapeDtypeStruct((B, NCH), jnp.float32),
    ),
)


def kernel(pcld_input, kpts_pre_input, cpt_pre_input, seg_pre_input,
           mesh_kpts_input):
    kflat = (kpts_pre_input.transpose(0, 3, 1, 2)
             .reshape(B, 3, 128, 128, 8)
             .transpose(0, 1, 2, 4, 3)
             .reshape(-1))
    pflat = (pcld_input.transpose(2, 0, 1)
             .reshape(3, 4, 8, 128, 128)
             .transpose(0, 1, 3, 2, 4)
             .reshape(-1))
    cflat = (cpt_pre_input.reshape(B, N, 3)
             .transpose(0, 2, 1)
             .reshape(-1))
    sflat = (seg_pre_input.reshape(B, 128, 128, 2)
             .transpose(0, 1, 3, 2)
             .reshape(-1))
    cands = _sc_topk(pflat, kflat, cflat, sflat).reshape(B, NCH, 3, 16)
    cx = cands[:, :, 0, :]
    cy = cands[:, :, 1, :]
    cz = cands[:, :, 2, :]
    mx = mesh_kpts_input[:, :, 0]
    my = mesh_kpts_input[:, :, 1]
    mz = mesh_kpts_input[:, :, 2]
    rflat, t, vx, vy, vz = _tc_pose(cx, cy, cz, mx, my, mz)
    batch_R = rflat.reshape(B, 3, 3)
    voted = jnp.stack([vx, vy, vz], axis=-1)
    return (batch_R, t, voted)
